# baseline (speedup 1.0000x reference)
import functools

import jax
import jax.numpy as jnp
from jax import lax
from jax.experimental import pallas as pl
from jax.experimental.pallas import tpu as pltpu
from jax.experimental.pallas import tpu_sc as plsc

_NC = 2
_NS = 16
_NW = _NC * _NS
_L = 16

_BSC = 2


def _sc_part(inputs, pos_table, BSC):
    B, S, D = inputs.shape

    CH = 8
    rows_per_w = S // _NW
    n_chunks = rows_per_w // CH
    vecs_per_row = D // _L

    mesh = plsc.VectorSubcoreMesh(core_axis_name="c", subcore_axis_name="s")

    @functools.partial(
        pl.kernel,
        mesh=mesh,
        out_type=jax.ShapeDtypeStruct((BSC, S, D), jnp.float32),
        scratch_types=[
            pltpu.VMEM((2, CH, D), jnp.float32),
            pltpu.VMEM((2, BSC, CH, D), jnp.float32),
            pltpu.SemaphoreType.DMA((2,)),
            pltpu.SemaphoreType.DMA((2, BSC)),
            pltpu.SemaphoreType.DMA((2, BSC)),
        ],
    )
    def sc_add(in_hbm, pos_hbm, out_hbm, pos_v, data_v, psem, isem, osem):
        wid = lax.axis_index("c") * _NS + lax.axis_index("s")
        base = wid * rows_per_w

        def pos_desc(c, q):
            row0 = base + c * CH
            return pltpu.make_async_copy(
                pos_hbm.at[pl.ds(row0, CH)], pos_v.at[q], psem.at[q]
            )

        def in_desc(c, p, b):
            row0 = base + c * CH
            return pltpu.make_async_copy(
                in_hbm.at[b, pl.ds(row0, CH)], data_v.at[p, b], isem.at[p, b]
            )

        def out_desc(c, p, b):
            row0 = base + c * CH
            return pltpu.make_async_copy(
                data_v.at[p, b], out_hbm.at[b, pl.ds(row0, CH)], osem.at[p, b]
            )

        pos_desc(0, 0).start()
        for b in range(BSC):
            in_desc(0, 0, b).start()

        def do_chunk(c, p):
            pos_desc(c, p).wait()

            @pl.when(c + 1 < n_chunks)
            def _():
                pos_desc(c + 1, 1 - p).start()

            for b in range(BSC):
                @pl.when(c >= 1)
                def _(_b=b):
                    out_desc(c - 1, 1 - p, _b).wait()

                @pl.when(c + 1 < n_chunks)
                def _(_b=b):
                    in_desc(c + 1, 1 - p, _b).start()

            for b in range(BSC):
                in_desc(c, p, b).wait()

                @plsc.parallel_loop(0, CH * vecs_per_row, step=1, unroll=8)
                def _(i, _b=b):
                    r = i // vecs_per_row
                    col = (i % vecs_per_row) * _L
                    plsc.addupdate(
                        data_v.at[p, _b, r, pl.ds(col, _L)],
                        pos_v[p, r, pl.ds(col, _L)],
                    )

                out_desc(c, p, b).start()

        def pair_body(g, carry):
            do_chunk(2 * g, 0)
            do_chunk(2 * g + 1, 1)
            return carry

        lax.fori_loop(0, n_chunks // 2, pair_body, 0)

        for b in range(BSC):
            out_desc(n_chunks - 1, 1, b).wait()

    return sc_add(inputs, pos_table)


def _tc_body(in_ref, pos_ref, out_ref):
    out_ref[...] = in_ref[...] + pos_ref[...]


def _tc_part(inputs, pos_table, b0, btc):
    B, S, D = inputs.shape
    SBLK = 512
    grid = (S // SBLK, btc)
    return pl.pallas_call(
        _tc_body,
        grid=grid,
        in_specs=[
            pl.BlockSpec((1, SBLK, D), lambda s, b: (b0 + b, s, 0)),
            pl.BlockSpec((1, SBLK, D), lambda s, b: (0, s, 0)),
        ],
        out_specs=pl.BlockSpec((1, SBLK, D), lambda s, b: (b, s, 0)),
        out_shape=jax.ShapeDtypeStruct((btc, S, D), jnp.float32),
    )(inputs, pos_table[None])


def kernel(inputs, pos_table):
    B, S, D = inputs.shape
    sc_out = _sc_part(inputs, pos_table, _BSC)
    tc_out = _tc_part(inputs, pos_table, _BSC, B - _BSC)
    return jnp.concatenate([sc_out, tc_out], axis=0)


# speedup vs baseline: 1.6886x; 1.6886x over previous
import functools

import jax
import jax.numpy as jnp
from jax import lax
from jax.experimental import pallas as pl
from jax.experimental.pallas import tpu as pltpu
from jax.experimental.pallas import tpu_sc as plsc

_NC = 2
_NS = 16
_NW = _NC * _NS
_L = 16


def kernel(inputs, pos_table):
    B, S, D = inputs.shape

    CH = 8
    rows_per_w = S // _NW
    n_chunks = rows_per_w // CH
    vecs_per_row = D // _L

    mesh = plsc.VectorSubcoreMesh(core_axis_name="c", subcore_axis_name="s")

    @functools.partial(
        pl.kernel,
        mesh=mesh,
        out_type=jax.ShapeDtypeStruct((B, S, D), jnp.float32),
        scratch_types=[
            pltpu.VMEM((2, CH, D), jnp.float32),
            pltpu.VMEM((2, B, CH, D), jnp.float32),
            pltpu.SemaphoreType.DMA((2,)),
            pltpu.SemaphoreType.DMA((2, B)),
            pltpu.SemaphoreType.DMA((2, B)),
        ],
    )
    def sc_add(in_hbm, pos_hbm, out_hbm, pos_v, data_v, psem, isem, osem):
        wid = lax.axis_index("s") * _NC + lax.axis_index("c")
        base = wid * rows_per_w

        def pos_desc(c, q):
            row0 = base + c * CH
            return pltpu.make_async_copy(
                pos_hbm.at[pl.ds(row0, CH)], pos_v.at[q], psem.at[q]
            )

        def in_desc(c, p, b):
            row0 = base + c * CH
            return pltpu.make_async_copy(
                in_hbm.at[b, pl.ds(row0, CH)], data_v.at[p, b], isem.at[p, b]
            )

        def out_desc(c, p, b):
            row0 = base + c * CH
            return pltpu.make_async_copy(
                data_v.at[p, b], out_hbm.at[b, pl.ds(row0, CH)], osem.at[p, b]
            )

        pos_desc(0, 0).start()
        for b in range(B):
            in_desc(0, 0, b).start()

        def do_chunk(c, p):
            pos_desc(c, p).wait()

            @pl.when(c + 1 < n_chunks)
            def _():
                pos_desc(c + 1, 1 - p).start()

            for b in range(B):
                @pl.when(c >= 1)
                def _(_b=b):
                    out_desc(c - 1, 1 - p, _b).wait()

                @pl.when(c + 1 < n_chunks)
                def _(_b=b):
                    in_desc(c + 1, 1 - p, _b).start()

            for b in range(B):
                in_desc(c, p, b).wait()

                @plsc.parallel_loop(0, CH * vecs_per_row, step=1, unroll=8)
                def _(i, _b=b):
                    r = i // vecs_per_row
                    col = (i % vecs_per_row) * _L
                    plsc.addupdate(
                        data_v.at[p, _b, r, pl.ds(col, _L)],
                        pos_v[p, r, pl.ds(col, _L)],
                    )

                out_desc(c, p, b).start()

        def pair_body(g, carry):
            do_chunk(2 * g, 0)
            do_chunk(2 * g + 1, 1)
            return carry

        lax.fori_loop(0, n_chunks // 2, pair_body, 0)

        for b in range(B):
            out_desc(n_chunks - 1, 1, b).wait()

    return sc_add(inputs, pos_table)
